# baseline (speedup 1.0000x reference)
import functools
import jax
import jax.numpy as jnp
from jax import lax
from jax.experimental import pallas as pl
from jax.experimental.pallas import tpu as pltpu
import jax.experimental.pallas.tpu_sc as plsc

N = 10000
E = 320000
D = 128
DE = 16
HID = 32
HEADS = 4
DEPTH = 5

NP = 10240
ROWS_PER_TILE = NP // 16
BR = 512
DUMMY = N

E_FULL = E + N
E_PAD = 331776
E_PRE = 321536

EB = 512
NEB = E_PAD // EB
NBLK = NP // EB

F32 = jnp.float32
I32 = jnp.int32


def _ae_body(ea_ref, fold_ref, o_ref):
    i = pl.program_id(0)
    m = jnp.dot(ea_ref[...], fold_ref[...], preferred_element_type=F32)
    rows = i * BR + lax.broadcasted_iota(I32, (BR, 32), 0)
    lanes = lax.broadcasted_iota(I32, (BR, 32), 1)
    o_ref[...] = m + jnp.where((lanes == 20) & (rows < E), 1.0, 0.0)


def _ae_matmul(ea_pad, fold32):
    return pl.pallas_call(
        _ae_body,
        grid=(E_PRE // BR,),
        in_specs=[pl.BlockSpec((BR, 16), lambda i: (i, 0)),
                  pl.BlockSpec((16, 32), lambda i: (0, 0))],
        out_specs=pl.BlockSpec((BR, 32), lambda i: (i, 0)),
        out_shape=jax.ShapeDtypeStruct((E_PRE, 32), F32),
    )(ea_pad, fold32)


def _prep_body(ae_hbm, fd_hbm, zer_hbm, out_hbm, rows_v, idx_v, acc_sh):
    c = lax.axis_index("c")
    s = lax.axis_index("s")
    pltpu.sync_copy(zer_hbm, acc_sh.at[pl.ds(s * ROWS_PER_TILE, ROWS_PER_TILE)])
    plsc.subcore_barrier()
    eh = E_PRE // 2
    et = eh // 16
    base = c * eh + s * et

    def body(g, carry):
        off = base + g * 64
        pltpu.sync_copy(fd_hbm.at[pl.ds(off, 64)], idx_v)
        pltpu.sync_copy(ae_hbm.at[pl.ds(off, 64)], rows_v)
        pltpu.sync_copy(rows_v, acc_sh.at[idx_v], add=True)
        return carry

    lax.fori_loop(0, et // 64, body, 0)
    plsc.subcore_barrier()
    r0 = s * ROWS_PER_TILE
    pltpu.sync_copy(acc_sh.at[pl.ds(r0, ROWS_PER_TILE)],
                    out_hbm.at[pl.ds(c * NP + r0, ROWS_PER_TILE)])


def _prep_call(aeones, fd_pre, zer32):
    mesh = plsc.VectorSubcoreMesh(core_axis_name="c", subcore_axis_name="s",
                                  num_cores=2, num_subcores=16)
    f = pl.kernel(
        _prep_body,
        out_type=jax.ShapeDtypeStruct((2 * NP, 32), F32),
        mesh=mesh,
        scratch_types=[pltpu.VMEM((64, 32), F32),
                       pltpu.VMEM((64,), I32),
                       pltpu.VMEM_SHARED((NP, 32), F32)],
        compiler_params=pltpu.CompilerParams(needs_layout_passes=False),
    )
    return f(aeones, fd_pre, zer32)


def _proj_body(h_ref, w_ref, wsd_ref, xs_ref, asad_ref):
    h = h_ref[...]
    xs_ref[...] = jnp.dot(h, w_ref[...], preferred_element_type=F32)
    asad_ref[...] = jnp.dot(h, wsd_ref[...], preferred_element_type=F32)


def _proj(h, w, wsd):
    return pl.pallas_call(
        _proj_body,
        grid=(NP // BR,),
        in_specs=[pl.BlockSpec((BR, D), lambda i: (i, 0)),
                  pl.BlockSpec((D, D), lambda i: (0, 0)),
                  pl.BlockSpec((D, 32), lambda i: (0, 0))],
        out_specs=[pl.BlockSpec((BR, D), lambda i: (i, 0)),
                   pl.BlockSpec((BR, 32), lambda i: (i, 0))],
        out_shape=[jax.ShapeDtypeStruct((NP, D), F32),
                   jax.ShapeDtypeStruct((NP, 32), F32)],
    )(h, w, wsd)


def _edge_body(lcol, ei_ref, ae_ref, xs_ref, asad_ref, mb_ref,
               num_ref, den_ref):
    i = pl.program_id(0)

    @pl.when(i == 0)
    def _init():
        num_ref[...] = jnp.zeros((NP, D), F32)
        den_ref[...] = jnp.zeros((NP, 32), F32)

    fs_row = ei_ref[0, 0:1, :]
    fd_row = ei_ref[0, 1:2, :]
    ae_blk = ae_ref[...]
    mbv = mb_ref[...]

    cn = lax.dot_general
    xs_g = jnp.zeros((EB, D), F32)
    asg = jnp.zeros((EB, 32), F32)
    adg = jnp.zeros((EB, 32), F32)
    iota_col = lax.broadcasted_iota(I32, (EB, EB), 0)
    fs_b = jnp.broadcast_to(fs_row, (EB, EB))
    fd_b = jnp.broadcast_to(fd_row, (EB, EB))
    for nb in range(NBLK):
        ohsT = (iota_col + nb * EB == fs_b).astype(F32)
        ohdT = (iota_col + nb * EB == fd_b).astype(F32)
        xs_blk = xs_ref[nb * EB:(nb + 1) * EB, :]
        asad_blk = asad_ref[nb * EB:(nb + 1) * EB, :]
        xs_g = xs_g + cn(ohsT, xs_blk, (((0,), (0,)), ((), ())),
                         preferred_element_type=F32)
        asg = asg + cn(ohsT, asad_blk, (((0,), (0,)), ((), ())),
                       preferred_element_type=F32)
        adg = adg + cn(ohdT, asad_blk, (((0,), (0,)), ((), ())),
                       preferred_element_type=F32)
    ps = []
    for h in range(HEADS):
        ae_h = ae_blk[:, lcol + h:lcol + h + 1]
        z = asg[:, h:h + 1] + adg[:, 4 + h:5 + h] + ae_h
        a = jnp.maximum(z, 0.2 * z)
        c0 = adg[:, 4 + h:5 + h] + mbv[0:1, h:h + 1]
        cc = jnp.maximum(c0, 0.2 * c0)
        ps.append(jnp.exp(a - cc))
    p128 = jnp.concatenate(
        [jnp.broadcast_to(p, (EB, HID)) for p in ps], axis=1)
    msg = xs_g * p128
    p32 = jnp.concatenate(ps + [jnp.zeros((EB, 28), F32)], axis=1)
    for nb in range(NBLK):
        ohd = (iota_col + nb * EB == fd_b).astype(F32)
        sl = slice(nb * EB, (nb + 1) * EB)
        num_ref[sl, :] += jnp.dot(ohd, msg, preferred_element_type=F32)
        den_ref[sl, :] += jnp.dot(ohd, p32, preferred_element_type=F32)


def _edge_call(ei8, aeL, xs, asad, mb, lcol):
    return pl.pallas_call(
        functools.partial(_edge_body, lcol),
        grid=(NEB,),
        in_specs=[pl.BlockSpec((1, 8, EB), lambda i: (i, 0, 0)),
                  pl.BlockSpec((EB, 128), lambda i: (i, 0)),
                  pl.BlockSpec((NP, D), lambda i: (0, 0)),
                  pl.BlockSpec((NP, 32), lambda i: (0, 0)),
                  pl.BlockSpec((1, 128), lambda i: (0, 0))],
        out_specs=[pl.BlockSpec((NP, D), lambda i: (0, 0)),
                   pl.BlockSpec((NP, 32), lambda i: (0, 0))],
        out_shape=[jax.ShapeDtypeStruct((NP, D), F32),
                   jax.ShapeDtypeStruct((NP, 32), F32)],
    )(ei8, aeL, xs, asad, mb)


def _epi_body(a_ref, d_ref, b_ref, o_ref):
    i = pl.program_id(0)
    acc = a_ref[...]
    den = d_ref[...]
    parts = []
    for h in range(HEADS):
        dh = den[:, h:h + 1] + 1e-16
        parts.append(acc[:, h * HID:(h + 1) * HID] / dh)
    hcat = jnp.concatenate(parts, axis=1) + b_ref[...]
    hcat = jnp.maximum(hcat, 0.0)
    rows = i * BR + lax.broadcasted_iota(I32, (BR, D), 0)
    o_ref[...] = jnp.where(rows < N, hcat, 0.0)


def _epilogue(acc, den, bi):
    return pl.pallas_call(
        _epi_body,
        grid=(NP // BR,),
        in_specs=[pl.BlockSpec((BR, D), lambda i: (i, 0)),
                  pl.BlockSpec((BR, 32), lambda i: (i, 0)),
                  pl.BlockSpec((1, D), lambda i: (0, 0))],
        out_specs=pl.BlockSpec((BR, D), lambda i: (i, 0)),
        out_shape=jax.ShapeDtypeStruct((NP, D), F32),
    )(acc, den, bi.reshape(1, D))


def _fc_body(x_ref, h_ref, w1_ref, w2_ref, b_ref, o_ref):
    o = (jnp.dot(x_ref[...], w1_ref[...], preferred_element_type=F32)
         + jnp.dot(h_ref[...], w2_ref[...], preferred_element_type=F32)
         + b_ref[...])
    o_ref[...] = jnp.maximum(o, 0.0)


def _final_fc(x_pad, h, fc_w, fc_b):
    return pl.pallas_call(
        _fc_body,
        grid=(NP // BR,),
        in_specs=[pl.BlockSpec((BR, D), lambda i: (i, 0)),
                  pl.BlockSpec((BR, D), lambda i: (i, 0)),
                  pl.BlockSpec((D, HID), lambda i: (0, 0)),
                  pl.BlockSpec((D, HID), lambda i: (0, 0)),
                  pl.BlockSpec((1, HID), lambda i: (0, 0))],
        out_specs=pl.BlockSpec((BR, HID), lambda i: (i, 0)),
        out_shape=jax.ShapeDtypeStruct((NP, HID), F32),
    )(x_pad, h, fc_w[:D], fc_w[D:], fc_b.reshape(1, HID))


def kernel(x, edge_index, edge_attr, W, att_src, att_dst, We, att_edge,
           b, fc_w, fc_b):
    src = edge_index[0]
    dst = edge_index[1]

    foldAll = (We.reshape(DEPTH, DE, HEADS, HID)
               * att_edge[:, None]).sum(-1)
    foldAll = foldAll.transpose(1, 0, 2).reshape(DE, DEPTH * HEADS)
    fold32 = jnp.pad(foldAll, ((0, 0), (0, 32 - DEPTH * HEADS)))
    waS = (W.reshape(DEPTH, D, HEADS, HID) * att_src[:, None]).sum(-1)
    waD = (W.reshape(DEPTH, D, HEADS, HID) * att_dst[:, None]).sum(-1)
    wSD = jnp.concatenate([waS, waD], axis=-1)
    wSD = jnp.pad(wSD, ((0, 0), (0, 0), (0, 24)))

    x = jnp.nan_to_num(x, nan=0.0, posinf=1000.0, neginf=-1000.0)
    x_pad = jnp.pad(x, ((0, NP - N), (0, 0)))
    ea_pad = jnp.pad(edge_attr, ((0, E_PRE - E), (0, 0)))
    fd_pre = jnp.pad(dst, (0, E_PRE - E), constant_values=DUMMY)

    zer32 = jnp.zeros((ROWS_PER_TILE, 32), F32)

    aeones = _ae_matmul(ea_pad, fold32)
    sums2 = _prep_call(aeones, fd_pre, zer32)
    sums = sums2[:NP] + sums2[NP:]
    deg = jnp.maximum(sums[:, 20:21], 1.0)
    ae_loop = sums[:, :DEPTH * HEADS] / deg

    loop_idx = jnp.arange(N, dtype=I32)
    pad_idx = jnp.full((E_PAD - E_FULL,), DUMMY, I32)
    fs = jnp.concatenate([src, loop_idx, pad_idx])
    fd = jnp.concatenate([dst, loop_idx, pad_idx])
    zsl = jnp.zeros((NEB, EB), I32)
    ei8 = jnp.stack([fs.reshape(NEB, EB), fd.reshape(NEB, EB)]
                    + [zsl] * 6, axis=1)

    ae_real = aeones[:E, :DEPTH * HEADS]
    ae_full = jnp.concatenate(
        [ae_real, ae_loop[:N],
         jnp.zeros((E_PAD - E_FULL, DEPTH * HEADS), F32)], axis=0)
    aeL = jnp.pad(ae_full, ((0, 0), (0, 128 - DEPTH * HEADS)))

    me = jnp.max(ae_full.reshape(E_PAD, DEPTH, HEADS), axis=0)

    h = x_pad
    for i in range(DEPTH):
        xs, asad = _proj(h, W[i], wSD[i])
        ms = jnp.max(asad[:, :HEADS], axis=0)
        mb = jnp.pad((ms + me[i])[None, :], ((0, 0), (0, 128 - HEADS)))
        num, den = _edge_call(ei8, aeL, xs, asad, mb, i * HEADS)
        h = _epilogue(num, den, b[i])

    out = _final_fc(x_pad, h, fc_w, fc_b)
    return out[:N]


# speedup vs baseline: 1.0254x; 1.0254x over previous
import functools
import jax
import jax.numpy as jnp
from jax import lax
from jax.experimental import pallas as pl
from jax.experimental.pallas import tpu as pltpu
import jax.experimental.pallas.tpu_sc as plsc

N = 10000
E = 320000
D = 128
DE = 16
HID = 32
HEADS = 4
DEPTH = 5

NP = 10240
ROWS_PER_TILE = NP // 16
BR = 512
DUMMY = N

E_FULL = E + N
E_PAD = 331776
E_PRE = 321536

EB = 512
NEB = E_PAD // EB
NBLK = NP // EB

F32 = jnp.float32
I32 = jnp.int32


def _ae_body(ea_ref, fold_ref, o_ref):
    i = pl.program_id(0)
    m = jnp.dot(ea_ref[...], fold_ref[...], preferred_element_type=F32)
    rows = i * BR + lax.broadcasted_iota(I32, (BR, 32), 0)
    lanes = lax.broadcasted_iota(I32, (BR, 32), 1)
    o_ref[...] = m + jnp.where((lanes == 20) & (rows < E), 1.0, 0.0)


def _ae_matmul(ea_pad, fold32):
    return pl.pallas_call(
        _ae_body,
        grid=(E_PRE // BR,),
        in_specs=[pl.BlockSpec((BR, 16), lambda i: (i, 0)),
                  pl.BlockSpec((16, 32), lambda i: (0, 0))],
        out_specs=pl.BlockSpec((BR, 32), lambda i: (i, 0)),
        out_shape=jax.ShapeDtypeStruct((E_PRE, 32), F32),
    )(ea_pad, fold32)


def _prep_body(ae_hbm, fd_hbm, zer_hbm, out_hbm, rows_v, idx_v, acc_sh):
    c = lax.axis_index("c")
    s = lax.axis_index("s")
    pltpu.sync_copy(zer_hbm, acc_sh.at[pl.ds(s * ROWS_PER_TILE, ROWS_PER_TILE)])
    plsc.subcore_barrier()
    eh = E_PRE // 2
    et = eh // 16
    base = c * eh + s * et

    def body(g, carry):
        off = base + g * 64
        pltpu.sync_copy(fd_hbm.at[pl.ds(off, 64)], idx_v)
        pltpu.sync_copy(ae_hbm.at[pl.ds(off, 64)], rows_v)
        pltpu.sync_copy(rows_v, acc_sh.at[idx_v], add=True)
        return carry

    lax.fori_loop(0, et // 64, body, 0)
    plsc.subcore_barrier()
    r0 = s * ROWS_PER_TILE
    pltpu.sync_copy(acc_sh.at[pl.ds(r0, ROWS_PER_TILE)],
                    out_hbm.at[pl.ds(c * NP + r0, ROWS_PER_TILE)])


def _prep_call(aeones, fd_pre, zer32):
    mesh = plsc.VectorSubcoreMesh(core_axis_name="c", subcore_axis_name="s",
                                  num_cores=2, num_subcores=16)
    f = pl.kernel(
        _prep_body,
        out_type=jax.ShapeDtypeStruct((2 * NP, 32), F32),
        mesh=mesh,
        scratch_types=[pltpu.VMEM((64, 32), F32),
                       pltpu.VMEM((64,), I32),
                       pltpu.VMEM_SHARED((NP, 32), F32)],
        compiler_params=pltpu.CompilerParams(needs_layout_passes=False),
    )
    return f(aeones, fd_pre, zer32)


def _proj_body(h_ref, w_ref, wsd_ref, xs_ref, asad_ref):
    h = h_ref[...]
    xs_ref[...] = jnp.dot(h, w_ref[...], preferred_element_type=F32)
    asad_ref[...] = jnp.dot(h, wsd_ref[...], preferred_element_type=F32)


def _proj(h, w, wsd):
    return pl.pallas_call(
        _proj_body,
        grid=(NP // BR,),
        in_specs=[pl.BlockSpec((BR, D), lambda i: (i, 0)),
                  pl.BlockSpec((D, D), lambda i: (0, 0)),
                  pl.BlockSpec((D, 32), lambda i: (0, 0))],
        out_specs=[pl.BlockSpec((BR, D), lambda i: (i, 0)),
                   pl.BlockSpec((BR, 32), lambda i: (i, 0))],
        out_shape=[jax.ShapeDtypeStruct((NP, D), F32),
                   jax.ShapeDtypeStruct((NP, 32), F32)],
    )(h, w, wsd)


def _edge_body(lcol, ei_ref, ae_ref, xs_ref, asad_ref, mb_ref,
               num_ref, den_ref):
    i = pl.program_id(0)

    @pl.when(i == 0)
    def _init():
        num_ref[...] = jnp.zeros((NP, D), F32)
        den_ref[...] = jnp.zeros((NP, 32), F32)

    fs_row = ei_ref[0, 0:1, :]
    fd_row = ei_ref[0, 1:2, :]
    ae_blk = ae_ref[...]
    mbv = mb_ref[...]

    cn = lax.dot_general
    xs_g = jnp.zeros((EB, D), F32)
    asg = jnp.zeros((EB, 32), F32)
    adg = jnp.zeros((EB, 32), F32)
    iota_col = lax.broadcasted_iota(I32, (EB, EB), 0)
    fs_b = jnp.broadcast_to(fs_row, (EB, EB))
    fd_b = jnp.broadcast_to(fd_row, (EB, EB))
    BF16 = jnp.bfloat16
    for nb in range(NBLK):
        ohsT = (iota_col + nb * EB == fs_b).astype(BF16)
        ohdT = (iota_col + nb * EB == fd_b).astype(BF16)
        xs_blk = xs_ref[nb * EB:(nb + 1) * EB, :].astype(BF16)
        asad_blk = asad_ref[nb * EB:(nb + 1) * EB, :].astype(BF16)
        xs_g = xs_g + cn(ohsT, xs_blk, (((0,), (0,)), ((), ())),
                         preferred_element_type=F32)
        asg = asg + cn(ohsT, asad_blk, (((0,), (0,)), ((), ())),
                       preferred_element_type=F32)
        adg = adg + cn(ohdT, asad_blk, (((0,), (0,)), ((), ())),
                       preferred_element_type=F32)
    ps = []
    for h in range(HEADS):
        ae_h = ae_blk[:, lcol + h:lcol + h + 1]
        z = asg[:, h:h + 1] + adg[:, 4 + h:5 + h] + ae_h
        a = jnp.maximum(z, 0.2 * z)
        c0 = adg[:, 4 + h:5 + h] + mbv[0:1, h:h + 1]
        cc = jnp.maximum(c0, 0.2 * c0)
        ps.append(jnp.exp(a - cc))
    p128 = jnp.concatenate(
        [jnp.broadcast_to(p, (EB, HID)) for p in ps], axis=1)
    msg = xs_g * p128
    p32 = jnp.concatenate(ps + [jnp.zeros((EB, 28), F32)], axis=1)
    msg16 = msg.astype(BF16)
    p3216 = p32.astype(BF16)
    for nb in range(NBLK):
        ohd = (iota_col + nb * EB == fd_b).astype(BF16)
        sl = slice(nb * EB, (nb + 1) * EB)
        num_ref[sl, :] += jnp.dot(ohd, msg16, preferred_element_type=F32)
        den_ref[sl, :] += jnp.dot(ohd, p3216, preferred_element_type=F32)


def _edge_call(ei8, aeL, xs, asad, mb, lcol):
    return pl.pallas_call(
        functools.partial(_edge_body, lcol),
        grid=(NEB,),
        in_specs=[pl.BlockSpec((1, 8, EB), lambda i: (i, 0, 0)),
                  pl.BlockSpec((EB, 128), lambda i: (i, 0)),
                  pl.BlockSpec((NP, D), lambda i: (0, 0)),
                  pl.BlockSpec((NP, 32), lambda i: (0, 0)),
                  pl.BlockSpec((1, 128), lambda i: (0, 0))],
        out_specs=[pl.BlockSpec((NP, D), lambda i: (0, 0)),
                   pl.BlockSpec((NP, 32), lambda i: (0, 0))],
        out_shape=[jax.ShapeDtypeStruct((NP, D), F32),
                   jax.ShapeDtypeStruct((NP, 32), F32)],
    )(ei8, aeL, xs, asad, mb)


def _epi_body(a_ref, d_ref, b_ref, o_ref):
    i = pl.program_id(0)
    acc = a_ref[...]
    den = d_ref[...]
    parts = []
    for h in range(HEADS):
        dh = den[:, h:h + 1] + 1e-16
        parts.append(acc[:, h * HID:(h + 1) * HID] / dh)
    hcat = jnp.concatenate(parts, axis=1) + b_ref[...]
    hcat = jnp.maximum(hcat, 0.0)
    rows = i * BR + lax.broadcasted_iota(I32, (BR, D), 0)
    o_ref[...] = jnp.where(rows < N, hcat, 0.0)


def _epilogue(acc, den, bi):
    return pl.pallas_call(
        _epi_body,
        grid=(NP // BR,),
        in_specs=[pl.BlockSpec((BR, D), lambda i: (i, 0)),
                  pl.BlockSpec((BR, 32), lambda i: (i, 0)),
                  pl.BlockSpec((1, D), lambda i: (0, 0))],
        out_specs=pl.BlockSpec((BR, D), lambda i: (i, 0)),
        out_shape=jax.ShapeDtypeStruct((NP, D), F32),
    )(acc, den, bi.reshape(1, D))


def _fc_body(x_ref, h_ref, w1_ref, w2_ref, b_ref, o_ref):
    o = (jnp.dot(x_ref[...], w1_ref[...], preferred_element_type=F32)
         + jnp.dot(h_ref[...], w2_ref[...], preferred_element_type=F32)
         + b_ref[...])
    o_ref[...] = jnp.maximum(o, 0.0)


def _final_fc(x_pad, h, fc_w, fc_b):
    return pl.pallas_call(
        _fc_body,
        grid=(NP // BR,),
        in_specs=[pl.BlockSpec((BR, D), lambda i: (i, 0)),
                  pl.BlockSpec((BR, D), lambda i: (i, 0)),
                  pl.BlockSpec((D, HID), lambda i: (0, 0)),
                  pl.BlockSpec((D, HID), lambda i: (0, 0)),
                  pl.BlockSpec((1, HID), lambda i: (0, 0))],
        out_specs=pl.BlockSpec((BR, HID), lambda i: (i, 0)),
        out_shape=jax.ShapeDtypeStruct((NP, HID), F32),
    )(x_pad, h, fc_w[:D], fc_w[D:], fc_b.reshape(1, HID))


def kernel(x, edge_index, edge_attr, W, att_src, att_dst, We, att_edge,
           b, fc_w, fc_b):
    src = edge_index[0]
    dst = edge_index[1]

    foldAll = (We.reshape(DEPTH, DE, HEADS, HID)
               * att_edge[:, None]).sum(-1)
    foldAll = foldAll.transpose(1, 0, 2).reshape(DE, DEPTH * HEADS)
    fold32 = jnp.pad(foldAll, ((0, 0), (0, 32 - DEPTH * HEADS)))
    waS = (W.reshape(DEPTH, D, HEADS, HID) * att_src[:, None]).sum(-1)
    waD = (W.reshape(DEPTH, D, HEADS, HID) * att_dst[:, None]).sum(-1)
    wSD = jnp.concatenate([waS, waD], axis=-1)
    wSD = jnp.pad(wSD, ((0, 0), (0, 0), (0, 24)))

    x = jnp.nan_to_num(x, nan=0.0, posinf=1000.0, neginf=-1000.0)
    x_pad = jnp.pad(x, ((0, NP - N), (0, 0)))
    ea_pad = jnp.pad(edge_attr, ((0, E_PRE - E), (0, 0)))
    fd_pre = jnp.pad(dst, (0, E_PRE - E), constant_values=DUMMY)

    zer32 = jnp.zeros((ROWS_PER_TILE, 32), F32)

    aeones = _ae_matmul(ea_pad, fold32)
    sums2 = _prep_call(aeones, fd_pre, zer32)
    sums = sums2[:NP] + sums2[NP:]
    deg = jnp.maximum(sums[:, 20:21], 1.0)
    ae_loop = sums[:, :DEPTH * HEADS] / deg

    loop_idx = jnp.arange(N, dtype=I32)
    pad_idx = jnp.full((E_PAD - E_FULL,), DUMMY, I32)
    fs = jnp.concatenate([src, loop_idx, pad_idx])
    fd = jnp.concatenate([dst, loop_idx, pad_idx])
    zsl = jnp.zeros((NEB, EB), I32)
    ei8 = jnp.stack([fs.reshape(NEB, EB), fd.reshape(NEB, EB)]
                    + [zsl] * 6, axis=1)

    ae_real = aeones[:E, :DEPTH * HEADS]
    ae_full = jnp.concatenate(
        [ae_real, ae_loop[:N],
         jnp.zeros((E_PAD - E_FULL, DEPTH * HEADS), F32)], axis=0)
    aeL = jnp.pad(ae_full, ((0, 0), (0, 128 - DEPTH * HEADS)))

    me = jnp.max(ae_full.reshape(E_PAD, DEPTH, HEADS), axis=0)

    h = x_pad
    for i in range(DEPTH):
        xs, asad = _proj(h, W[i], wSD[i])
        ms = jnp.max(asad[:, :HEADS], axis=0)
        mb = jnp.pad((ms + me[i])[None, :], ((0, 0), (0, 128 - HEADS)))
        num, den = _edge_call(ei8, aeL, xs, asad, mb, i * HEADS)
        h = _epilogue(num, den, b[i])

    out = _final_fc(x_pad, h, fc_w, fc_b)
    return out[:N]
